# baseline (speedup 1.0000x reference)
import functools
import math

import jax
import jax.numpy as jnp
from jax import lax
from jax.experimental import pallas as pl
from jax.experimental.pallas import tpu as pltpu
from jax.experimental.pallas import tpu_sc as plsc

BATCH = 1024
SEQ = 512
D = 128
N = BATCH * SEQ
NUM_WORKERS = 32
ROWS_PER_W = N // NUM_WORKERS
CHUNK = 32
NCHUNKS = ROWS_PER_W // CHUNK
NBUF = 8
GLEAD = 2
WLAG = 4
LANES = 16
SCALE = math.sqrt(D)


def _adapter_sc(ids_flat, embed_table, pos_table):
    mesh = plsc.VectorSubcoreMesh(core_axis_name="c", subcore_axis_name="s")

    @functools.partial(
        pl.kernel,
        mesh=mesh,
        out_type=jax.ShapeDtypeStruct((N, D), jnp.float32),
        scratch_types=(
            [pltpu.VMEM((ROWS_PER_W,), jnp.int32),
             pltpu.VMEM((SEQ, D), jnp.float32)]
            + [pltpu.VMEM((CHUNK, D), jnp.float32)] * NBUF
            + [pltpu.SemaphoreType.DMA] * (2 * NBUF)
        ),
    )
    def k(ids_hbm, table_hbm, pos_hbm, out_hbm, idx_v, pos_v, *ring):
        bufs = ring[:NBUF]
        gsems = ring[NBUF:2 * NBUF]
        osems = ring[2 * NBUF:]
        wid = lax.axis_index("s") * 2 + lax.axis_index("c")
        base = wid * ROWS_PER_W

        pltpu.sync_copy(ids_hbm.at[pl.ds(base, ROWS_PER_W)], idx_v)
        pltpu.sync_copy(pos_hbm, pos_v)

        def gather(cc, buf, sem):
            return pltpu.make_async_copy(
                table_hbm.at[idx_v.at[pl.ds(cc * CHUNK, CHUNK)]], buf, sem)

        def writeback(cc, buf, sem):
            return pltpu.make_async_copy(
                buf, out_hbm.at[pl.ds(base + cc * CHUNK, CHUNK)], sem)

        def compute(cc, buf):
            pos_row = lax.rem(cc * CHUNK, SEQ)

            @plsc.parallel_loop(0, CHUNK, unroll=2)
            def _(r):
                g = [buf[r, pl.ds(c0, LANES)] for c0 in range(0, D, LANES)]
                p = [pos_v[pos_row + r, pl.ds(c0, LANES)]
                     for c0 in range(0, D, LANES)]
                for i, c0 in enumerate(range(0, D, LANES)):
                    buf[r, pl.ds(c0, LANES)] = g[i] * SCALE + p[i]

        for w in range(GLEAD):
            gather(w, bufs[w], gsems[w]).start()

        @pl.loop(0, NCHUNKS, step=NBUF)
        def _(c):
            for j in range(NBUF):
                cc = c + j
                bg = (j + GLEAD) % NBUF

                bw = (j - WLAG) % NBUF

                @pl.when(cc >= WLAG)
                def _():
                    writeback(cc - WLAG, bufs[bw], osems[bw]).wait()

                @pl.when(cc + GLEAD < NCHUNKS)
                def _():
                    gather(cc + GLEAD, bufs[bg], gsems[bg]).start()

                gather(cc, bufs[j], gsems[j]).wait()
                compute(cc, bufs[j])
                writeback(cc, bufs[j], osems[j]).start()

        for t in range(WLAG):
            cc = NCHUNKS - WLAG + t
            writeback(cc, bufs[cc % NBUF], osems[cc % NBUF]).wait()

    return k(ids_flat, embed_table, pos_table)


def kernel(input_ids, embed_table, pos_table):
    ids_flat = input_ids.reshape(N).astype(jnp.int32)
    out = _adapter_sc(ids_flat, embed_table, pos_table)
    return out.reshape(BATCH, SEQ, D)


# speedup vs baseline: 1.2168x; 1.2168x over previous
import functools
import math

import jax
import jax.numpy as jnp
from jax import lax
from jax.experimental import pallas as pl
from jax.experimental.pallas import tpu as pltpu
from jax.experimental.pallas import tpu_sc as plsc

BATCH = 1024
SEQ = 512
D = 128
N = BATCH * SEQ
NUM_WORKERS = 32
ROWS_PER_W = N // NUM_WORKERS
CHUNK = 64
NCHUNKS = ROWS_PER_W // CHUNK
NBUF = 4
LANES = 16
SCALE = math.sqrt(D)


def _adapter_sc(ids_flat, embed_table, pos_table):
    mesh = plsc.VectorSubcoreMesh(core_axis_name="c", subcore_axis_name="s")

    @functools.partial(
        pl.kernel,
        mesh=mesh,
        out_type=jax.ShapeDtypeStruct((N, D), jnp.float32),
        scratch_types=(
            [pltpu.VMEM((ROWS_PER_W,), jnp.int32),
             pltpu.VMEM((SEQ, D), jnp.float32)]
            + [pltpu.VMEM((CHUNK, D), jnp.float32)] * NBUF
            + [pltpu.SemaphoreType.DMA] * (2 * NBUF)
        ),
    )
    def k(ids_hbm, table_hbm, pos_hbm, out_hbm, idx_v, pos_v, *ring):
        bufs = ring[:NBUF]
        gsems = ring[NBUF:2 * NBUF]
        osems = ring[2 * NBUF:]
        wid = lax.axis_index("s") * 2 + lax.axis_index("c")
        base = wid * ROWS_PER_W

        pltpu.sync_copy(ids_hbm.at[pl.ds(base, ROWS_PER_W)], idx_v)
        pltpu.sync_copy(pos_hbm, pos_v)

        def gather(cc, buf, sem):
            return pltpu.make_async_copy(
                table_hbm.at[idx_v.at[pl.ds(cc * CHUNK, CHUNK)]], buf, sem)

        def writeback(cc, buf, sem):
            return pltpu.make_async_copy(
                buf, out_hbm.at[pl.ds(base + cc * CHUNK, CHUNK)], sem)

        def compute(cc, buf):
            pos_row = lax.rem(cc * CHUNK, SEQ)

            @plsc.parallel_loop(0, CHUNK, unroll=2)
            def _(r):
                g = [buf[r, pl.ds(c0, LANES)] for c0 in range(0, D, LANES)]
                p = [pos_v[pos_row + r, pl.ds(c0, LANES)]
                     for c0 in range(0, D, LANES)]
                for i, c0 in enumerate(range(0, D, LANES)):
                    buf[r, pl.ds(c0, LANES)] = g[i] * SCALE + p[i]

        gather(0, bufs[0], gsems[0]).start()
        gather(1, bufs[1], gsems[1]).start()

        @pl.loop(0, NCHUNKS, step=NBUF)
        def _(c):
            for j in range(NBUF):
                cc = c + j
                b2 = (j + 2) % NBUF

                @pl.when(cc >= 2)
                def _():
                    writeback(cc - 2, bufs[b2], osems[b2]).wait()

                @pl.when(cc + 2 < NCHUNKS)
                def _():
                    gather(cc + 2, bufs[b2], gsems[b2]).start()

                gather(cc, bufs[j], gsems[j]).wait()
                compute(cc, bufs[j])
                writeback(cc, bufs[j], osems[j]).start()

        writeback(NCHUNKS - 2, bufs[(NCHUNKS - 2) % NBUF],
                  osems[(NCHUNKS - 2) % NBUF]).wait()
        writeback(NCHUNKS - 1, bufs[(NCHUNKS - 1) % NBUF],
                  osems[(NCHUNKS - 1) % NBUF]).wait()

    return k(ids_flat, embed_table, pos_table)


def kernel(input_ids, embed_table, pos_table):
    ids_flat = input_ids.reshape(N).astype(jnp.int32)
    out = _adapter_sc(ids_flat, embed_table, pos_table)
    return out.reshape(BATCH, SEQ, D)
